# baseline (speedup 1.0000x reference)
import functools

import jax
import jax.numpy as jnp
from jax import lax
from jax.experimental import pallas as pl
from jax.experimental.pallas import tpu as pltpu
from jax.experimental.pallas import tpu_sc as plsc

_N = 10000
_E = 320000
_D = 128
_H = 128
_C = 64

_NC = 2
_NS = 16
_NW = _NC * _NS
_EPW = _E // _NW
_K = 64
_NCH = 160
_NBUF = 4
_EPWP = _NCH * _K
_EP = _NW * _EPWP
_SUP = 16
_NSUP = _NCH // _SUP
_NP = 10240
_RPT = _NP // _NS


def _make_sc_segsum(with_counts: bool):
    mesh = plsc.VectorSubcoreMesh(core_axis_name="c", subcore_axis_name="s")
    out_type = [jax.ShapeDtypeStruct((2 * _NP, _D), jnp.float32)]
    scratch = (
        [pltpu.VMEM((_K,), jnp.int32) for _ in range(_NBUF)]
        + [pltpu.VMEM((_K,), jnp.int32) for _ in range(_NBUF)]
        + [pltpu.VMEM((_K, _D), jnp.float32) for _ in range(_NBUF)]
        + [pltpu.VMEM_SHARED((_NP, _D), jnp.float32)]
        + [pltpu.SemaphoreType.DMA for _ in range(2 * _NBUF)]
    )
    if with_counts:
        out_type.append(jax.ShapeDtypeStruct((2 * _NP, _D), jnp.float32))
        out_type.append(jax.ShapeDtypeStruct((2 * _NS * _NP,), jnp.float32))
        scratch.extend([
            pltpu.VMEM((_NP,), jnp.float32),
            pltpu.VMEM((_RPT,), jnp.float32),
            pltpu.VMEM((_RPT,), jnp.float32),
            pltpu.VMEM((16, _D), jnp.float32),
        ])

    def body(*refs):
        if with_counts:
            (src_hbm, dst_hbm, tab_hbm, zrow_hbm, zhist_hbm,
             sum_out, cnt_out, hists_out, *rest) = refs
        else:
            (src_hbm, dst_hbm, tab_hbm, zrow_hbm, sum_out, *rest) = refs
        sidxs = rest[0:_NBUF]
        didxs = rest[_NBUF:2 * _NBUF]
        rowss = rest[2 * _NBUF:3 * _NBUF]
        acc = rest[3 * _NBUF]
        sems = rest[3 * _NBUF + 1:4 * _NBUF + 1]
        ssems = rest[4 * _NBUF + 1:5 * _NBUF + 1]
        if with_counts:
            hist, tmp, accum, repl = rest[5 * _NBUF + 1:]
        cid = lax.axis_index("c")
        sid = lax.axis_index("s")
        wid = cid * _NS + sid

        pltpu.sync_copy(zrow_hbm, acc.at[pl.ds(sid * _RPT, _RPT)])
        if with_counts:
            pltpu.sync_copy(zhist_hbm, hist)
        plsc.subcore_barrier()
        ones16 = jnp.ones((16,), jnp.float32)

        def stage_and_fire(j, r):
            chunk = wid * _NCH + j
            pltpu.sync_copy(src_hbm.at[pl.ds(chunk * _K, _K)], sidxs[r])
            pltpu.sync_copy(dst_hbm.at[pl.ds(chunk * _K, _K)], didxs[r])
            pltpu.async_copy(tab_hbm.at[sidxs[r]], rowss[r], sems[r])

        def consume(r):
            pltpu.make_async_copy(tab_hbm.at[sidxs[r]], rowss[r], sems[r]).wait()
            pltpu.async_copy(rowss[r], acc.at[didxs[r]], ssems[r], add=True)
            if with_counts:
                for i in range(_K // 16):
                    v = didxs[r][pl.ds(i * 16, 16)]
                    plsc.addupdate_scatter(hist, [v], ones16)

        def wait_scatter(r):
            pltpu.make_async_copy(rowss[r], acc.at[didxs[r]], ssems[r]).wait()

        for r in range(_NBUF):
            stage_and_fire(r, r)

        def step(g, carry):
            for r in range(_NBUF):
                consume(r)
            for r in range(_NBUF):
                wait_scatter(r)
                nxt = jnp.minimum(_NBUF * g + r + _NBUF, _NCH - 1)
                stage_and_fire(nxt, r)
            return carry

        lax.fori_loop(0, _NCH // _NBUF, step, 0)
        for r in range(_NBUF):
            pltpu.make_async_copy(tab_hbm.at[sidxs[r]], rowss[r], sems[r]).wait()
        plsc.subcore_barrier()

        dst0 = cid * _NP + sid * _RPT
        pltpu.sync_copy(acc.at[pl.ds(sid * _RPT, _RPT)],
                        sum_out.at[pl.ds(dst0, _RPT)])

        if with_counts:
            pltpu.sync_copy(hist, hists_out.at[pl.ds(wid * _NP, _NP)])
            plsc.subcore_barrier()
            base = cid * _NS * _NP + sid * _RPT
            pltpu.sync_copy(hists_out.at[pl.ds(base, _RPT)], accum)

            def red(r, carry):
                pltpu.sync_copy(hists_out.at[pl.ds(base + r * _NP, _RPT)], tmp)
                for g in range(_RPT // 16):
                    sl = pl.ds(g * 16, 16)
                    accum[sl] = accum[sl] + tmp[sl]
                return carry

            lax.fori_loop(1, _NS, red, 0)

            def repl_slab(g, carry):
                for i in range(16):
                    v = plsc.load_gather(
                        accum, [jnp.full((16,), g * 16 + i, jnp.int32)])
                    for f in range(_D // 16):
                        repl[i, pl.ds(f * 16, 16)] = v
                pltpu.sync_copy(repl, cnt_out.at[pl.ds(dst0 + g * 16, 16)])
                return carry

            lax.fori_loop(0, _RPT // 16, repl_slab, 0)

    kwargs = {}
    if with_counts:
        kwargs["compiler_params"] = pltpu.CompilerParams(
            needs_layout_passes=False)
    return pl.kernel(body, mesh=mesh, out_type=out_type,
                     scratch_types=scratch, **kwargs)


_sc_sum_cnt = _make_sc_segsum(with_counts=True)
_sc_sum = _make_sc_segsum(with_counts=False)

_BLK = 1000
_GRID = _N // _BLK


def _tc_a_body(x_ref, wl_ref, wr_ref, bl_ref, y_ref, z_ref):
    x = x_ref[...]
    y_ref[...] = jnp.dot(x, wl_ref[...], preferred_element_type=jnp.float32)
    z_ref[...] = jnp.dot(x, wr_ref[...], preferred_element_type=jnp.float32) + bl_ref[...]


def _tc_b_body(sa_ref, sb_ref, ca_ref, cb_ref, z_ref, wl_ref, wr_ref, bl_ref,
               y_ref, z2_ref):
    cnt = jnp.maximum(ca_ref[...] + cb_ref[...], 1.0)
    h = jnp.maximum((sa_ref[...] + sb_ref[...]) / cnt + z_ref[...], 0.0)
    y_ref[...] = jnp.dot(h, wl_ref[...], preferred_element_type=jnp.float32)
    z2_ref[...] = jnp.dot(h, wr_ref[...], preferred_element_type=jnp.float32) + bl_ref[...]


def _tc_c_body(sa_ref, sb_ref, ca_ref, cb_ref, z_ref, wc_ref, bc_ref,
               logits_ref, h_ref):
    cnt = jnp.maximum(ca_ref[...] + cb_ref[...], 1.0)
    h = jnp.maximum((sa_ref[...] + sb_ref[...]) / cnt + z_ref[...], 0.0)
    h_ref[...] = h
    logits_ref[...] = jnp.dot(h, wc_ref[...], preferred_element_type=jnp.float32) + bc_ref[...]


def _row_spec(w):
    return pl.BlockSpec((_BLK, w), lambda i: (i, 0))


def _full_spec(h, w):
    return pl.BlockSpec((h, w), lambda i: (0, 0))


_tc_a = pl.pallas_call(
    _tc_a_body,
    grid=(_GRID,),
    in_specs=[_row_spec(_D), _full_spec(_D, _H), _full_spec(_D, _H), _full_spec(1, _H)],
    out_specs=[_row_spec(_H), _row_spec(_H)],
    out_shape=[jax.ShapeDtypeStruct((_N, _H), jnp.float32)] * 2,
)

_tc_b = pl.pallas_call(
    _tc_b_body,
    grid=(_GRID,),
    in_specs=[_row_spec(_H), _row_spec(_H), _row_spec(_H), _row_spec(_H),
              _row_spec(_H), _full_spec(_H, _H), _full_spec(_H, _H), _full_spec(1, _H)],
    out_specs=[_row_spec(_H), _row_spec(_H)],
    out_shape=[jax.ShapeDtypeStruct((_N, _H), jnp.float32)] * 2,
)

_tc_c = pl.pallas_call(
    _tc_c_body,
    grid=(_GRID,),
    in_specs=[_row_spec(_H), _row_spec(_H), _row_spec(_H), _row_spec(_H),
              _row_spec(_H), _full_spec(_H, _C), _full_spec(1, _C)],
    out_specs=[pl.BlockSpec((_BLK, _C), lambda i: (i, 0)), _row_spec(_H)],
    out_shape=[jax.ShapeDtypeStruct((_N, _C), jnp.float32),
               jax.ShapeDtypeStruct((_N, _H), jnp.float32)],
)


def kernel(x_doc, x_token, edge_index, edge_weight,
           W_l1, b_l1, W_r1, W_l2, b_l2, W_r2, W_c, b_c):
    pad = _EP - _E
    src_p = jnp.concatenate([edge_index[0], jnp.zeros((pad,), jnp.int32)])
    dst_p = jnp.concatenate([edge_index[1], jnp.full((pad,), _N, jnp.int32)])
    src_r = src_p
    dst_r = dst_p
    zrow = jnp.zeros((_RPT, _D), jnp.float32)
    zhist = jnp.zeros((_NP,), jnp.float32)

    y1, z1 = _tc_a(x_doc, W_l1, W_r1, b_l1.reshape(1, _H))
    sums1, cnts, _hists = _sc_sum_cnt(src_r, dst_r, y1, zrow, zhist)
    ca, cb = cnts[:_N], cnts[_NP:_NP + _N]
    y2, z2 = _tc_b(sums1[:_N], sums1[_NP:_NP + _N], ca, cb, z1,
                   W_l2, W_r2, b_l2.reshape(1, _H))
    (sums2,) = _sc_sum(src_r, dst_r, y2, zrow)
    logits, h2 = _tc_c(sums2[:_N], sums2[_NP:_NP + _N], ca, cb, z2,
                       W_c, b_c.reshape(1, _C))
    return (logits, h2)


# speedup vs baseline: 1.1767x; 1.1767x over previous
import functools

import jax
import jax.numpy as jnp
from jax import lax
from jax.experimental import pallas as pl
from jax.experimental.pallas import tpu as pltpu
from jax.experimental.pallas import tpu_sc as plsc

_N = 10000
_E = 320000
_D = 128
_H = 128
_C = 64

_NC = 2
_NS = 16
_NW = _NC * _NS
_EPW = _E // _NW
_K = 128
_NCH = 80
_NCH0 = 116
_NCH1 = 44
_NBUF = 2
_EPWP = _NCH * _K
_EP = _NW * _EPWP
_SUP = 16
_NSUP = _NCH // _SUP
_NP = 10240
_RPT = _NP // _NS


def _make_sc_segsum(with_counts: bool):
    mesh = plsc.VectorSubcoreMesh(core_axis_name="c", subcore_axis_name="s")
    out_type = [jax.ShapeDtypeStruct((2 * _NP, _D), jnp.float32)]
    scratch = (
        [pltpu.VMEM((_K,), jnp.int32) for _ in range(_NBUF)]
        + [pltpu.VMEM((_K,), jnp.int32) for _ in range(_NBUF)]
        + [pltpu.VMEM((_K, _D), jnp.float32) for _ in range(_NBUF)]
        + [pltpu.VMEM_SHARED((_NP, _D), jnp.float32)]
        + [pltpu.SemaphoreType.DMA for _ in range(_NBUF)]
    )
    if with_counts:
        out_type.append(jax.ShapeDtypeStruct((2 * _NP, _D), jnp.float32))
        out_type.append(jax.ShapeDtypeStruct((2 * _NS * _NP,), jnp.float32))
        scratch.extend([
            pltpu.VMEM((_NP,), jnp.float32),
            pltpu.VMEM((_RPT,), jnp.float32),
            pltpu.VMEM((_RPT,), jnp.float32),
            pltpu.VMEM((16, _D), jnp.float32),
        ])

    def body(*refs):
        if with_counts:
            (src_hbm, dst_hbm, tab_hbm, zrow_hbm, zhist_hbm,
             sum_out, cnt_out, hists_out, *rest) = refs
        else:
            (src_hbm, dst_hbm, tab_hbm, zrow_hbm, sum_out, *rest) = refs
        sidxs = rest[0:_NBUF]
        didxs = rest[_NBUF:2 * _NBUF]
        rowss = rest[2 * _NBUF:3 * _NBUF]
        acc = rest[3 * _NBUF]
        sems = rest[3 * _NBUF + 1:4 * _NBUF + 1]
        if with_counts:
            hist, tmp, accum, repl = rest[4 * _NBUF + 1:]
        cid = lax.axis_index("c")
        sid = lax.axis_index("s")
        wid = cid * _NS + sid

        pltpu.sync_copy(zrow_hbm, acc.at[pl.ds(sid * _RPT, _RPT)])
        if with_counts:
            pltpu.sync_copy(zhist_hbm, hist)
        plsc.subcore_barrier()
        ones16 = jnp.ones((16,), jnp.float32)

        nch = jnp.where(cid == 0, _NCH0, _NCH1)
        base = jnp.where(cid == 0, sid * _NCH0, _NS * _NCH0 + sid * _NCH1)

        def stage_and_fire(j, r):
            chunk = base + j
            pltpu.sync_copy(src_hbm.at[pl.ds(chunk * _K, _K)], sidxs[r])
            pltpu.sync_copy(dst_hbm.at[pl.ds(chunk * _K, _K)], didxs[r])
            pltpu.async_copy(tab_hbm.at[sidxs[r]], rowss[r], sems[r])

        def consume(r):
            pltpu.make_async_copy(tab_hbm.at[sidxs[r]], rowss[r], sems[r]).wait()
            pltpu.sync_copy(rowss[r], acc.at[didxs[r]], add=True)
            if with_counts:
                for i in range(_K // 16):
                    v = didxs[r][pl.ds(i * 16, 16)]
                    plsc.addupdate_scatter(hist, [v], ones16)

        stage_and_fire(0, 0)

        def step(g, carry):
            stage_and_fire(2 * g + 1, 1)
            consume(0)
            stage_and_fire(jnp.minimum(2 * g + 2, nch - 1), 0)
            consume(1)
            return carry

        lax.fori_loop(0, nch // 2, step, 0)
        pltpu.make_async_copy(tab_hbm.at[sidxs[0]], rowss[0], sems[0]).wait()
        plsc.subcore_barrier()

        dst0 = cid * _NP + sid * _RPT
        pltpu.sync_copy(acc.at[pl.ds(sid * _RPT, _RPT)],
                        sum_out.at[pl.ds(dst0, _RPT)])

        if with_counts:
            pltpu.sync_copy(hist, hists_out.at[pl.ds(wid * _NP, _NP)])
            plsc.subcore_barrier()
            base = cid * _NS * _NP + sid * _RPT
            pltpu.sync_copy(hists_out.at[pl.ds(base, _RPT)], accum)

            def red(r, carry):
                pltpu.sync_copy(hists_out.at[pl.ds(base + r * _NP, _RPT)], tmp)
                for g in range(_RPT // 16):
                    sl = pl.ds(g * 16, 16)
                    accum[sl] = accum[sl] + tmp[sl]
                return carry

            lax.fori_loop(1, _NS, red, 0)

            def repl_slab(g, carry):
                for i in range(16):
                    v = plsc.load_gather(
                        accum, [jnp.full((16,), g * 16 + i, jnp.int32)])
                    for f in range(_D // 16):
                        repl[i, pl.ds(f * 16, 16)] = v
                pltpu.sync_copy(repl, cnt_out.at[pl.ds(dst0 + g * 16, 16)])
                return carry

            lax.fori_loop(0, _RPT // 16, repl_slab, 0)

    kwargs = {}
    if with_counts:
        kwargs["compiler_params"] = pltpu.CompilerParams(
            needs_layout_passes=False)
    return pl.kernel(body, mesh=mesh, out_type=out_type,
                     scratch_types=scratch, **kwargs)


_sc_sum_cnt = _make_sc_segsum(with_counts=True)
_sc_sum = _make_sc_segsum(with_counts=False)

_BLK = 1000
_GRID = _N // _BLK


def _tc_a_body(x_ref, wl_ref, wr_ref, bl_ref, y_ref, z_ref):
    x = x_ref[...]
    y_ref[...] = jnp.dot(x, wl_ref[...], preferred_element_type=jnp.float32)
    z_ref[...] = jnp.dot(x, wr_ref[...], preferred_element_type=jnp.float32) + bl_ref[...]


def _tc_b_body(sa_ref, sb_ref, ca_ref, cb_ref, z_ref, wl_ref, wr_ref, bl_ref,
               y_ref, z2_ref):
    cnt = jnp.maximum(ca_ref[...] + cb_ref[...], 1.0)
    h = jnp.maximum((sa_ref[...] + sb_ref[...]) / cnt + z_ref[...], 0.0)
    y_ref[...] = jnp.dot(h, wl_ref[...], preferred_element_type=jnp.float32)
    z2_ref[...] = jnp.dot(h, wr_ref[...], preferred_element_type=jnp.float32) + bl_ref[...]


def _tc_c_body(sa_ref, sb_ref, ca_ref, cb_ref, z_ref, wc_ref, bc_ref,
               logits_ref, h_ref):
    cnt = jnp.maximum(ca_ref[...] + cb_ref[...], 1.0)
    h = jnp.maximum((sa_ref[...] + sb_ref[...]) / cnt + z_ref[...], 0.0)
    h_ref[...] = h
    logits_ref[...] = jnp.dot(h, wc_ref[...], preferred_element_type=jnp.float32) + bc_ref[...]


def _row_spec(w):
    return pl.BlockSpec((_BLK, w), lambda i: (i, 0))


def _full_spec(h, w):
    return pl.BlockSpec((h, w), lambda i: (0, 0))


_tc_a = pl.pallas_call(
    _tc_a_body,
    grid=(_GRID,),
    in_specs=[_row_spec(_D), _full_spec(_D, _H), _full_spec(_D, _H), _full_spec(1, _H)],
    out_specs=[_row_spec(_H), _row_spec(_H)],
    out_shape=[jax.ShapeDtypeStruct((_N, _H), jnp.float32)] * 2,
)

_tc_b = pl.pallas_call(
    _tc_b_body,
    grid=(_GRID,),
    in_specs=[_row_spec(_H), _row_spec(_H), _row_spec(_H), _row_spec(_H),
              _row_spec(_H), _full_spec(_H, _H), _full_spec(_H, _H), _full_spec(1, _H)],
    out_specs=[_row_spec(_H), _row_spec(_H)],
    out_shape=[jax.ShapeDtypeStruct((_N, _H), jnp.float32)] * 2,
)

_tc_c = pl.pallas_call(
    _tc_c_body,
    grid=(_GRID,),
    in_specs=[_row_spec(_H), _row_spec(_H), _row_spec(_H), _row_spec(_H),
              _row_spec(_H), _full_spec(_H, _C), _full_spec(1, _C)],
    out_specs=[pl.BlockSpec((_BLK, _C), lambda i: (i, 0)), _row_spec(_H)],
    out_shape=[jax.ShapeDtypeStruct((_N, _C), jnp.float32),
               jax.ShapeDtypeStruct((_N, _H), jnp.float32)],
)


def kernel(x_doc, x_token, edge_index, edge_weight,
           W_l1, b_l1, W_r1, W_l2, b_l2, W_r2, W_c, b_c):
    pad = _EP - _E
    src_p = jnp.concatenate([edge_index[0], jnp.zeros((pad,), jnp.int32)])
    dst_p = jnp.concatenate([edge_index[1], jnp.full((pad,), _N, jnp.int32)])
    src_r = src_p
    dst_r = dst_p
    zrow = jnp.zeros((_RPT, _D), jnp.float32)
    zhist = jnp.zeros((_NP,), jnp.float32)

    y1, z1 = _tc_a(x_doc, W_l1, W_r1, b_l1.reshape(1, _H))
    sums1, cnts, _hists = _sc_sum_cnt(src_r, dst_r, y1, zrow, zhist)
    ca, cb = cnts[:_N], cnts[_NP:_NP + _N]
    y2, z2 = _tc_b(sums1[:_N], sums1[_NP:_NP + _N], ca, cb, z1,
                   W_l2, W_r2, b_l2.reshape(1, _H))
    (sums2,) = _sc_sum(src_r, dst_r, y2, zrow)
    logits, h2 = _tc_c(sums2[:_N], sums2[_NP:_NP + _N], ca, cb, z2,
                       W_c, b_c.reshape(1, _C))
    return (logits, h2)
